# baseline (speedup 1.0000x reference)
import functools

import jax
import jax.numpy as jnp
from jax import lax
from jax.experimental import pallas as pl
from jax.experimental.pallas import tpu as pltpu
from jax.experimental.pallas import tpu_sc as plsc

B, L, K = 4096, 50, 8
VOCAB, D = 1000000, 32
NW = 32
BPW = B // NW
NIDX = BPW * K
NGATH = NIDX // 128
HALF = 16
DT = D // 8


BLK = 128
NFULL = VOCAB // BLK
TPW = NFULL // NW + 1
TAILC = VOCAB - NFULL * BLK
TLROWS = VOCAB * D // 128


def _make_detile_kernel():
  mesh = plsc.VectorSubcoreMesh(core_axis_name="c", subcore_axis_name="s")

  @functools.partial(
      pl.kernel,
      out_type=jax.ShapeDtypeStruct((TLROWS, 128), jnp.float32),
      mesh=mesh,
      scratch_types=[
          pltpu.VMEM((2, D, BLK), jnp.float32),
          pltpu.VMEM((2, D, BLK), jnp.float32),
          pltpu.VMEM((D, TAILC), jnp.float32),
          pltpu.VMEM((TAILC * D // 128, 128), jnp.float32),
          pltpu.SemaphoreType.DMA,
          pltpu.SemaphoreType.DMA,
      ],
      compiler_params=pltpu.CompilerParams(use_tc_tiling_on_sc=True,
                                           needs_layout_passes=False),
  )
  def sc_detile(tt_hbm, tl_hbm, in_v, out_v, tail_in, tail_out, sem0, sem1):
    wid = lax.axis_index("s") * 2 + lax.axis_index("c")
    sems = (sem0, sem1)
    lane = lax.iota(jnp.int32, 16)
    col_base = (lane & 3) << 5
    row_vecs = [(lane >> 2) + (j16 * 4) for j16 in range(8)]

    def fire(t, buf):
      blk = wid + NW * t
      @pl.when(blk < NFULL)
      def _():
        pltpu.async_copy(tt_hbm.at[:, pl.ds(blk * BLK, BLK)],
                         in_v.at[buf], sems[buf])

    def wait_and_compute(t, buf):
      blk = wid + NW * t
      @pl.when(blk < NFULL)
      def _():
        pltpu.make_async_copy(tt_hbm.at[:, pl.ds(0, BLK)],
                              in_v.at[buf], sems[buf]).wait()
        for j16 in range(8):
          for d in range(D):
            v = in_v[buf, d, pl.ds(j16 * 16, 16)]
            plsc.store_scatter(out_v.at[buf], [row_vecs[j16], col_base + d], v)
        pltpu.sync_copy(out_v.at[buf], tl_hbm.at[pl.ds(blk * D, D)])

    fire(0, 0)

    def pair_body(i, carry):
      t0 = i * 2
      fire(t0 + 1, 1)
      wait_and_compute(t0, 0)
      fire(t0 + 2, 0)
      wait_and_compute(t0 + 1, 1)
      return carry

    lax.fori_loop(0, (TPW + 1) // 2, pair_body, 0)

    @pl.when(wid == 0)
    def _():
      pltpu.sync_copy(tt_hbm.at[:, pl.ds(NFULL * BLK, TAILC)], tail_in)
      for j16 in range(TAILC // 16):
        for d in range(D):
          v = tail_in[d, pl.ds(j16 * 16, 16)]
          plsc.store_scatter(tail_out, [row_vecs[j16], col_base + d], v)
      pltpu.sync_copy(tail_out, tl_hbm.at[pl.ds(NFULL * D, TAILC * D // 128)])

  return sc_detile


def _make_sc_kernel():
  mesh = plsc.VectorSubcoreMesh(core_axis_name="c", subcore_axis_name="s")

  @functools.partial(
      pl.kernel,
      out_type=jax.ShapeDtypeStruct((L, DT, NW, 8, 128), jnp.float32),
      mesh=mesh,
      scratch_types=[
          pltpu.VMEM((2, K, BPW), jnp.int32),
          pltpu.VMEM((2, K, BPW, D), jnp.float32),
          pltpu.VMEM((2, D, BPW), jnp.float32),
          pltpu.SemaphoreType.DMA,
          pltpu.SemaphoreType.DMA,
      ],
      compiler_params=pltpu.CompilerParams(use_tc_tiling_on_sc=False,
                                           needs_layout_passes=False),
  )
  def sc_embed(x_hbm, table_hbm, out_hbm, idx_v, rows_v, out_v, sem0, sem1):
    wid = lax.axis_index("s") * 2 + lax.axis_index("c")
    b0 = wid * BPW
    sems = (sem0, sem1)
    lane = lax.iota(jnp.int32, 16)

    def stage_and_fire(l, buf):
      pltpu.sync_copy(x_hbm.at[l, :, pl.ds(b0, BPW)], idx_v.at[buf])
      for k in range(K):
        pltpu.async_copy(table_hbm.at[idx_v.at[buf, k]],
                         rows_v.at[buf, k], sems[buf])

    def drain(buf):
      for k in range(K):
        pltpu.make_async_copy(table_hbm.at[idx_v.at[buf, k]],
                              rows_v.at[buf, k], sems[buf]).wait()

    def compute_and_store(l, buf):
      def pos_body(p, carry2):
        for h in range(0, D, HALF):
          acc = rows_v[buf, 0, p, pl.ds(h, HALF)]
          for k in range(1, K):
            acc = acc + rows_v[buf, k, p, pl.ds(h, HALF)]
          plsc.store_scatter(out_v.at[buf],
                             [lane + h, jnp.full((16,), p, jnp.int32)], acc)
        return carry2

      lax.fori_loop(0, BPW, pos_body, 0, unroll=2)
      for dt in range(DT):
        pltpu.sync_copy(out_v.at[buf, pl.ds(dt * 8, 8)],
                        out_hbm.at[l, dt, wid])

    stage_and_fire(0, 0)

    def pair_body(i, carry):
      l0 = i * 2
      stage_and_fire(l0 + 1, 1)
      drain(0)
      compute_and_store(l0, 0)
      @pl.when(l0 + 2 < L)
      def _():
        stage_and_fire(l0 + 2, 0)
      drain(1)
      compute_and_store(l0 + 1, 1)
      return carry

    lax.fori_loop(0, L // 2, pair_body, 0)

  return sc_embed


_sc_detile = _make_detile_kernel()
_sc_embed = _make_sc_kernel()


@jax.jit
def kernel(x, table):
  tl = _sc_detile(jnp.transpose(table))
  out5 = _sc_embed(jnp.transpose(x, (1, 2, 0)), tl.reshape(VOCAB, D))
  return out5.transpose((2, 4, 0, 1, 3)).reshape(B, L, D)


# speedup vs baseline: 1.0516x; 1.0516x over previous
import functools

import jax
import jax.numpy as jnp
from jax import lax
from jax.experimental import pallas as pl
from jax.experimental.pallas import tpu as pltpu
from jax.experimental.pallas import tpu_sc as plsc

B, L, K = 4096, 50, 8
VOCAB, D = 1000000, 32
NW = 32
BPW = B // NW
NIDX = BPW * K
NGATH = NIDX // 128
HALF = 16
DT = D // 8


BLKC = 128
NFULL = VOCAB // BLKC
TPW = 246
TAILC = VOCAB - NFULL * BLKC
TLROWS = VOCAB * D // 128


def _make_detile_kernel():
  mesh = plsc.VectorSubcoreMesh(core_axis_name="c", subcore_axis_name="s")

  @functools.partial(
      pl.kernel,
      out_type=jax.ShapeDtypeStruct((TLROWS, 128), jnp.float32),
      mesh=mesh,
      scratch_types=[
          pltpu.VMEM((6, D, BLKC), jnp.float32),
          pltpu.VMEM((2, BLKC * D // 128, 128), jnp.float32),
          pltpu.VMEM((D, TAILC), jnp.float32),
          pltpu.VMEM((TAILC * D // 128, 128), jnp.float32),
      ] + [pltpu.SemaphoreType.DMA] * 8,
      compiler_params=pltpu.CompilerParams(use_tc_tiling_on_sc=True,
                                           needs_layout_passes=False),
  )
  def sc_detile(tt_hbm, tl_hbm, in_v, out_v, tail_in, tail_out,
                is0, is1, is2, is3, is4, is5, osem0, osem1):
    wid = lax.axis_index("s") * 2 + lax.axis_index("c")
    isems = (is0, is1, is2, is3, is4, is5)
    osems = (osem0, osem1)
    lane = lax.iota(jnp.int32, 16)
    col_base = (lane & 3) << 5
    row_base = lane >> 2
    ORORWS = BLKC * D // 128

    def fire(t, buf):
      blk = wid + NW * t
      @pl.when(blk < NFULL)
      def _():
        pltpu.async_copy(tt_hbm.at[:, pl.ds(blk * BLKC, BLKC)],
                         in_v.at[buf], isems[buf])

    def wait_and_compute(t, buf):
      blk = wid + NW * t
      obuf = buf % 2
      @pl.when(blk < NFULL)
      def _():
        pltpu.make_async_copy(tt_hbm.at[:, pl.ds(0, BLKC)],
                              in_v.at[buf], isems[buf]).wait()
        @pl.when(t >= 2)
        def _():
          pltpu.make_async_copy(out_v.at[obuf], tl_hbm.at[pl.ds(0, ORORWS)],
                                osems[obuf]).wait()

        def tr_body(j16, cc):
          rvec = row_base + j16 * 4
          for d in range(D):
            v = in_v[buf, d, pl.ds(j16 * 16, 16)]
            plsc.store_scatter(out_v.at[obuf], [rvec, col_base + d], v)
          return cc

        lax.fori_loop(0, BLKC // 16, tr_body, 0, unroll=2)
        pltpu.async_copy(out_v.at[obuf],
                         tl_hbm.at[pl.ds(blk * ORORWS, ORORWS)], osems[obuf])

    NB = 6
    for t in range(NB - 1):
      fire(t, t)

    def ring_body(g, carry):
      t0 = g * NB
      for b in range(NB):
        fire(t0 + b + NB - 1, (b + NB - 1) % NB)
        wait_and_compute(t0 + b, b)
      return carry

    lax.fori_loop(0, TPW // NB, ring_body, 0)

    for t in (TPW - 4, TPW - 3, TPW - 2, TPW - 1):
      blk = wid + NW * t
      @pl.when((blk < NFULL) & (blk + 2 * NW >= NFULL))
      def _(t=t):
        pltpu.make_async_copy(out_v.at[t % 2], tl_hbm.at[pl.ds(0, ORORWS)],
                              osems[t % 2]).wait()

    @pl.when(wid == 0)
    def _():
      pltpu.sync_copy(tt_hbm.at[:, pl.ds(NFULL * BLKC, TAILC)], tail_in)
      for j16 in range(TAILC // 16):
        for d in range(D):
          v = tail_in[d, pl.ds(j16 * 16, 16)]
          plsc.store_scatter(tail_out, [row_base + j16 * 4, col_base + d], v)
      pltpu.sync_copy(tail_out, tl_hbm.at[pl.ds(NFULL * D, TAILC * D // 128)])

  return sc_detile


def _make_sc_kernel():
  mesh = plsc.VectorSubcoreMesh(core_axis_name="c", subcore_axis_name="s")

  @functools.partial(
      pl.kernel,
      out_type=jax.ShapeDtypeStruct((L, DT, NW, 8, 128), jnp.float32),
      mesh=mesh,
      scratch_types=[
          pltpu.VMEM((2, K, BPW), jnp.int32),
          pltpu.VMEM((2, K, BPW, D), jnp.float32),
          pltpu.VMEM((2, D, BPW), jnp.float32),
          pltpu.SemaphoreType.DMA,
          pltpu.SemaphoreType.DMA,
          pltpu.SemaphoreType.DMA,
          pltpu.SemaphoreType.DMA,
          pltpu.SemaphoreType.DMA,
          pltpu.SemaphoreType.DMA,
      ],
      compiler_params=pltpu.CompilerParams(use_tc_tiling_on_sc=False,
                                           needs_layout_passes=False),
  )
  def sc_embed(x_hbm, table_hbm, out_hbm, idx_v, rows_v, out_v,
               gsem0, gsem1, isem0, isem1, osem0, osem1):
    wid = lax.axis_index("s") * 2 + lax.axis_index("c")
    b0 = wid * BPW
    gsems = (gsem0, gsem1)
    isems = (isem0, isem1)
    osems = (osem0, osem1)
    lane = lax.iota(jnp.int32, 16)

    def fire_idx(l, buf):
      @pl.when(l < L)
      def _():
        pltpu.async_copy(x_hbm.at[l, :, pl.ds(b0, BPW)], idx_v.at[buf],
                         isems[buf])

    def wait_idx_fire_gathers(l, buf):
      @pl.when(l < L)
      def _():
        pltpu.make_async_copy(x_hbm.at[0, :, pl.ds(b0, BPW)], idx_v.at[buf],
                              isems[buf]).wait()
        for k in range(K):
          pltpu.async_copy(table_hbm.at[idx_v.at[buf, k]],
                           rows_v.at[buf, k], gsems[buf])

    def drain_gathers(buf):
      for k in range(K):
        pltpu.make_async_copy(table_hbm.at[idx_v.at[buf, k]],
                              rows_v.at[buf, k], gsems[buf]).wait()

    def wait_outs(buf):
      for dt in range(DT):
        pltpu.make_async_copy(out_v.at[buf, pl.ds(dt * 8, 8)],
                              out_hbm.at[0, dt, wid], osems[buf]).wait()

    def compute(l, buf):
      def pos_body(p, carry2):
        for h in range(0, D, HALF):
          acc = rows_v[buf, 0, p, pl.ds(h, HALF)]
          for k in range(1, K):
            acc = acc + rows_v[buf, k, p, pl.ds(h, HALF)]
          plsc.store_scatter(out_v.at[buf],
                             [lane + h, jnp.full((16,), p, jnp.int32)], acc)
        return carry2

      lax.fori_loop(0, BPW, pos_body, 0, unroll=2)
      for dt in range(DT):
        pltpu.async_copy(out_v.at[buf, pl.ds(dt * 8, 8)],
                         out_hbm.at[l, dt, wid], osems[buf])

    def chunk_steps(l, buf):
      nb = 1 - buf
      drain_gathers(buf)
      fire_idx(l + 2, buf)
      wait_idx_fire_gathers(l + 1, nb)
      @pl.when(l >= 2)
      def _():
        wait_outs(buf)
      compute(l, buf)

    fire_idx(0, 0)
    wait_idx_fire_gathers(0, 0)
    fire_idx(1, 1)

    def pair_body(i, carry):
      l0 = i * 2
      chunk_steps(l0, 0)
      chunk_steps(l0 + 1, 1)
      return carry

    lax.fori_loop(0, L // 2, pair_body, 0)
    wait_outs(0)
    wait_outs(1)

  return sc_embed


_sc_detile = _make_detile_kernel()
_sc_embed = _make_sc_kernel()


@jax.jit
def kernel(x, table):
  tl = _sc_detile(jnp.transpose(table))
  out5 = _sc_embed(jnp.transpose(x, (1, 2, 0)), tl.reshape(VOCAB, D))
  return out5.transpose((2, 4, 0, 1, 3)).reshape(B, L, D)
